# baseline (speedup 1.0000x reference)
import jax
import jax.numpy as jnp
from jax import lax
from jax.experimental import pallas as pl
from jax.experimental.pallas import tpu as pltpu
from jax.experimental.pallas import tpu_sc as plsc

N_BINS = 64
EPS = 1e-06
LOW = 0.0
HIGH = 1.0

NC = 2
NS = 16
NW = NC * NS
LANES = 16

HALF = 256
NBUF = 4
BTC = HALF // 128


def _sc_tokenize(x_hbm, zeros_hbm, tokens_hbm, mask_hbm,
                 x_v, buf_v, colsave_v, ones_v, sems, mask_sem):
    M = x_hbm.shape[0]
    nslab = tokens_hbm.shape[0]
    bdim = x_hbm.shape[0] // nslab
    mw = M // NW
    spw = nslab // NW
    nch = spw * (bdim // HALF)
    wid = lax.axis_index("s") * NC + lax.axis_index("c")
    base = wid * mw

    pltpu.sync_copy(x_hbm.at[pl.ds(base, mw)], x_v)
    pltpu.sync_copy(zeros_hbm, buf_v)
    zvec_i = jnp.zeros((LANES,), jnp.int32)

    def zinit(i, carry):
        colsave_v[pl.ds(i * LANES, LANES)] = zvec_i
        return carry

    lax.fori_loop(0, (NBUF * HALF) // LANES, zinit, 0)

    iota = lax.broadcasted_iota(jnp.int32, (LANES,), 0)
    onef = jnp.ones((LANES,), jnp.float32)
    zerof = jnp.zeros((LANES,), jnp.float32)
    ngrp = HALF // LANES

    nh = bdim // HALF

    def do_chunk(c, slot):
        slab = wid * spw + c // nh
        bh = c % nh
        slotv = jnp.full((LANES,), slot, jnp.int32)

        @pl.when(c >= NBUF)
        def _():
            pltpu.make_async_copy(
                buf_v.at[slot],
                tokens_hbm.at[slab, :, pl.ds(bh * BTC, BTC)],
                sems.at[slot],
            ).wait()

        def grp(g, carry):
            bl = g * LANES + iota
            xv = x_v[pl.ds(c * HALF + g * LANES, LANES)]
            xv = jnp.minimum(jnp.maximum(xv, LOW + EPS), HIGH - EPS)
            bins = (xv * N_BINS).astype(jnp.int32)
            btl = bl >> 7
            b128 = bl & 127
            sbase = slot * HALF + g * LANES
            old = colsave_v[pl.ds(sbase, LANES)]
            plsc.store_scatter(
                buf_v, [slotv, old >> 3, btl, old & 7, b128], zerof)
            plsc.store_scatter(
                buf_v, [slotv, bins >> 3, btl, bins & 7, b128], onef)
            colsave_v[pl.ds(sbase, LANES)] = bins
            return carry

        lax.fori_loop(0, ngrp, grp, 0)
        pltpu.async_copy(
            buf_v.at[slot],
            tokens_hbm.at[slab, :, pl.ds(bh * BTC, BTC)],
            sems.at[slot],
        )

    def pair(c2, carry):
        for b in range(NBUF):
            do_chunk(c2 * NBUF + b, b)
        return carry

    lax.fori_loop(0, nch // NBUF, pair, 0)

    for b in range(NBUF):
        pltpu.make_async_copy(
            buf_v.at[b],
            tokens_hbm.at[wid * spw, :, pl.ds(0, BTC)],
            sems.at[b],
        ).wait()

    nones = ones_v.shape[0]

    def ofill(i, carry):
        ones_v[pl.ds(i * LANES, LANES)] = onef
        return carry

    lax.fori_loop(0, nones // LANES, ofill, 0)
    for q in range(mw // nones):
        pltpu.async_copy(
            ones_v, mask_hbm.at[pl.ds(base + q * nones, nones)], mask_sem
        ).wait()


@jax.jit
def kernel(observations):
    B, T, D = observations.shape
    M = B * T * D
    xt = jnp.transpose(observations, (1, 2, 0)).reshape(M)
    zeros = jnp.zeros((NBUF, 8, BTC, 8, 128), jnp.float32)
    mesh = plsc.VectorSubcoreMesh(core_axis_name="c", subcore_axis_name="s")
    mw = M // NW
    tokens6, mask = pl.kernel(
        _sc_tokenize,
        mesh=mesh,
        compiler_params=pltpu.CompilerParams(
            needs_layout_passes=False, use_tc_tiling_on_sc=True),
        out_type=[
            jax.ShapeDtypeStruct((T * D, 8, 8, 8, 128), jnp.float32),
            jax.ShapeDtypeStruct((M,), jnp.float32),
        ],
        scratch_types=[
            pltpu.VMEM((mw,), jnp.float32),
            pltpu.VMEM((NBUF, 8, BTC, 8, 128), jnp.float32),
            pltpu.VMEM((NBUF * HALF,), jnp.int32),
            pltpu.VMEM((4096,), jnp.float32),
            pltpu.SemaphoreType.DMA((NBUF,)),
            pltpu.SemaphoreType.DMA,
        ],
    )(xt, zeros)
    t6 = tokens6.reshape(T, D, 8, 8, 8, 128)
    tokens = jnp.transpose(t6, (3, 5, 0, 1, 2, 4)).reshape(B, T, D, N_BINS)
    maskt = jnp.transpose(mask.reshape(T, D, B), (2, 0, 1))
    return (tokens, maskt)


# speedup vs baseline: 1.0200x; 1.0200x over previous
import jax
import jax.numpy as jnp
from jax import lax
from jax.experimental import pallas as pl
from jax.experimental.pallas import tpu as pltpu
from jax.experimental.pallas import tpu_sc as plsc

N_BINS = 64
EPS = 1e-06
LOW = 0.0
HIGH = 1.0

NC = 2
NS = 16
NW = NC * NS
LANES = 16

HALF = 512
NBUF = 2


def _sc_tokenize(x_hbm, zeros_hbm, tokens_hbm, mask_hbm,
                 x_v, buf_v, colsave_v, ones_v, sems, mask_sem):
    M = x_hbm.shape[0]
    nslab = tokens_hbm.shape[0]
    bdim = x_hbm.shape[0] // nslab
    mw = M // NW
    spw = nslab // NW
    nch = spw * (bdim // HALF)
    wid = lax.axis_index("s") * NC + lax.axis_index("c")
    base = wid * mw

    pltpu.sync_copy(x_hbm.at[pl.ds(base, mw)], x_v)
    pltpu.sync_copy(zeros_hbm, buf_v)
    zvec_i = jnp.zeros((LANES,), jnp.int32)

    def zinit(i, carry):
        colsave_v[pl.ds(i * LANES, LANES)] = zvec_i
        return carry

    lax.fori_loop(0, (NBUF * HALF) // LANES, zinit, 0)

    onef_ = jnp.ones((LANES,), jnp.float32)
    nones = ones_v.shape[0]

    def ofill(i, carry):
        ones_v[pl.ds(i * LANES, LANES)] = onef_
        return carry

    lax.fori_loop(0, nones // LANES, ofill, 0)
    for q in range(mw // nones):
        pltpu.async_copy(
            ones_v, mask_hbm.at[pl.ds(base + q * nones, nones)], mask_sem
        )

    iota = lax.broadcasted_iota(jnp.int32, (LANES,), 0)
    onef = jnp.ones((LANES,), jnp.float32)
    zerof = jnp.zeros((LANES,), jnp.float32)
    ngrp = HALF // LANES

    def do_chunk(c, slot):
        slab = wid * spw + c // 2
        bh = c % 2
        slotv = jnp.full((LANES,), slot, jnp.int32)

        @pl.when(c >= NBUF)
        def _():
            pltpu.make_async_copy(
                buf_v.at[slot],
                tokens_hbm.at[slab, :, pl.ds(bh * 4, 4)],
                sems.at[slot],
            ).wait()

        def grp(g, carry):
            bl = g * LANES + iota
            xv = x_v[pl.ds(c * HALF + g * LANES, LANES)]
            xv = jnp.minimum(jnp.maximum(xv, LOW + EPS), HIGH - EPS)
            bins = (xv * N_BINS).astype(jnp.int32)
            btl = bl >> 7
            b128 = bl & 127
            sbase = slot * HALF + g * LANES
            old = colsave_v[pl.ds(sbase, LANES)]
            plsc.store_scatter(
                buf_v, [slotv, old >> 3, btl, old & 7, b128], zerof)
            plsc.store_scatter(
                buf_v, [slotv, bins >> 3, btl, bins & 7, b128], onef)
            colsave_v[pl.ds(sbase, LANES)] = bins
            return carry

        lax.fori_loop(0, ngrp, grp, 0)
        pltpu.async_copy(
            buf_v.at[slot],
            tokens_hbm.at[slab, :, pl.ds(bh * 4, 4)],
            sems.at[slot],
        )

    def pair(c2, carry):
        for b in range(NBUF):
            do_chunk(c2 * NBUF + b, b)
        return carry

    lax.fori_loop(0, nch // NBUF, pair, 0)

    for b in range(NBUF):
        pltpu.make_async_copy(
            buf_v.at[b],
            tokens_hbm.at[wid * spw, :, pl.ds(b * 4, 4)],
            sems.at[b],
        ).wait()

    for q in range(mw // nones):
        pltpu.make_async_copy(
            ones_v, mask_hbm.at[pl.ds(base + q * nones, nones)], mask_sem
        ).wait()


@jax.jit
def kernel(observations):
    B, T, D = observations.shape
    M = B * T * D
    xt = jnp.transpose(observations, (1, 2, 0)).reshape(M)
    zeros = jnp.zeros((NBUF, 8, 4, 8, 128), jnp.float32)
    mesh = plsc.VectorSubcoreMesh(core_axis_name="c", subcore_axis_name="s")
    mw = M // NW
    tokens6, mask = pl.kernel(
        _sc_tokenize,
        mesh=mesh,
        compiler_params=pltpu.CompilerParams(
            needs_layout_passes=False, use_tc_tiling_on_sc=True),
        out_type=[
            jax.ShapeDtypeStruct((T * D, 8, 8, 8, 128), jnp.float32),
            jax.ShapeDtypeStruct((M,), jnp.float32),
        ],
        scratch_types=[
            pltpu.VMEM((mw,), jnp.float32),
            pltpu.VMEM((NBUF, 8, 4, 8, 128), jnp.float32),
            pltpu.VMEM((NBUF * HALF,), jnp.int32),
            pltpu.VMEM((4096,), jnp.float32),
            pltpu.SemaphoreType.DMA((NBUF,)),
            pltpu.SemaphoreType.DMA,
        ],
    )(xt, zeros)
    t6 = tokens6.reshape(T, D, 8, 8, 8, 128)
    tokens = jnp.transpose(t6, (3, 5, 0, 1, 2, 4)).reshape(B, T, D, N_BINS)
    maskt = jnp.transpose(mask.reshape(T, D, B), (2, 0, 1))
    return (tokens, maskt)


# speedup vs baseline: 1.0333x; 1.0131x over previous
import jax
import jax.numpy as jnp
from jax import lax
from jax.experimental import pallas as pl
from jax.experimental.pallas import tpu as pltpu
from jax.experimental.pallas import tpu_sc as plsc

N_BINS = 64
EPS = 1e-06
LOW = 0.0
HIGH = 1.0

NC = 2
NS = 16
NW = NC * NS
LANES = 16

HALF = 512
NBUF = 2


def _sc_tokenize(x_hbm, zeros_hbm, tokens_hbm, mask_hbm,
                 x_v, buf_v, colsave_v, ones_v, sems, mask_sem):
    M = x_hbm.shape[0]
    nslab = tokens_hbm.shape[0]
    bdim = x_hbm.shape[0] // nslab
    mw = M // NW
    spw = nslab // NW
    nch = spw * (bdim // HALF)
    wid = lax.axis_index("s") * NC + lax.axis_index("c")
    base = wid * mw

    pltpu.async_copy(x_hbm.at[pl.ds(base, mw)], x_v, sems.at[0])
    pltpu.async_copy(zeros_hbm, buf_v, sems.at[1])
    zvec_i = jnp.zeros((LANES,), jnp.int32)

    def zinit(i, carry):
        colsave_v[pl.ds(i * LANES, LANES)] = zvec_i
        return carry

    lax.fori_loop(0, (NBUF * HALF) // LANES, zinit, 0)

    onef_ = jnp.ones((LANES,), jnp.float32)
    nones = ones_v.shape[0]

    def ofill(i, carry):
        ones_v[pl.ds(i * LANES, LANES)] = onef_
        return carry

    lax.fori_loop(0, nones // LANES, ofill, 0)
    for q in range(mw // nones):
        pltpu.async_copy(
            ones_v, mask_hbm.at[pl.ds(base + q * nones, nones)], mask_sem
        )
    pltpu.make_async_copy(x_hbm.at[pl.ds(base, mw)], x_v, sems.at[0]).wait()
    pltpu.make_async_copy(zeros_hbm, buf_v, sems.at[1]).wait()

    iota = lax.broadcasted_iota(jnp.int32, (LANES,), 0)
    onef = jnp.ones((LANES,), jnp.float32)
    zerof = jnp.zeros((LANES,), jnp.float32)
    ngrp = HALF // LANES

    def do_chunk(c, slot):
        slab = wid * spw + c // 2
        bh = c % 2
        slotv = jnp.full((LANES,), slot, jnp.int32)

        @pl.when(c >= NBUF)
        def _():
            pltpu.make_async_copy(
                buf_v.at[slot],
                tokens_hbm.at[slab, :, pl.ds(bh * 4, 4)],
                sems.at[slot],
            ).wait()

        def grp(g, carry):
            bl = g * LANES + iota
            xv = x_v[pl.ds(c * HALF + g * LANES, LANES)]
            xv = jnp.minimum(jnp.maximum(xv, LOW + EPS), HIGH - EPS)
            bins = (xv * N_BINS).astype(jnp.int32)
            btl = bl >> 7
            b128 = bl & 127
            sbase = slot * HALF + g * LANES
            old = colsave_v[pl.ds(sbase, LANES)]
            plsc.store_scatter(
                buf_v, [slotv, old >> 3, btl, old & 7, b128], zerof)
            plsc.store_scatter(
                buf_v, [slotv, bins >> 3, btl, bins & 7, b128], onef)
            colsave_v[pl.ds(sbase, LANES)] = bins
            return carry

        lax.fori_loop(0, ngrp, grp, 0)
        pltpu.async_copy(
            buf_v.at[slot],
            tokens_hbm.at[slab, :, pl.ds(bh * 4, 4)],
            sems.at[slot],
        )

    def pair(c2, carry):
        for b in range(NBUF):
            do_chunk(c2 * NBUF + b, b)
        return carry

    lax.fori_loop(0, nch // NBUF, pair, 0)

    for b in range(NBUF):
        pltpu.make_async_copy(
            buf_v.at[b],
            tokens_hbm.at[wid * spw, :, pl.ds(b * 4, 4)],
            sems.at[b],
        ).wait()

    for q in range(mw // nones):
        pltpu.make_async_copy(
            ones_v, mask_hbm.at[pl.ds(base + q * nones, nones)], mask_sem
        ).wait()


@jax.jit
def kernel(observations):
    B, T, D = observations.shape
    M = B * T * D
    xt = jnp.transpose(observations, (1, 2, 0)).reshape(M)
    zeros = jnp.zeros((NBUF, 8, 4, 8, 128), jnp.float32)
    mesh = plsc.VectorSubcoreMesh(core_axis_name="c", subcore_axis_name="s")
    mw = M // NW
    tokens6, mask = pl.kernel(
        _sc_tokenize,
        mesh=mesh,
        compiler_params=pltpu.CompilerParams(
            needs_layout_passes=False, use_tc_tiling_on_sc=True),
        out_type=[
            jax.ShapeDtypeStruct((T * D, 8, 8, 8, 128), jnp.float32),
            jax.ShapeDtypeStruct((M,), jnp.float32),
        ],
        scratch_types=[
            pltpu.VMEM((mw,), jnp.float32),
            pltpu.VMEM((NBUF, 8, 4, 8, 128), jnp.float32),
            pltpu.VMEM((NBUF * HALF,), jnp.int32),
            pltpu.VMEM((4096,), jnp.float32),
            pltpu.SemaphoreType.DMA((NBUF,)),
            pltpu.SemaphoreType.DMA,
        ],
    )(xt, zeros)
    t6 = tokens6.reshape(T, D, 8, 8, 8, 128)
    tokens = jnp.transpose(t6, (3, 5, 0, 1, 2, 4)).reshape(B, T, D, N_BINS)
    maskt = jnp.transpose(mask.reshape(T, D, B), (2, 0, 1))
    return (tokens, maskt)
